# Initial kernel scaffold; baseline (speedup 1.0000x reference)
import jax
import jax.numpy as jnp
from jax.experimental import pallas as pl


def kernel(x, edge_index, W1, b1, W2, b2, W3, b3):
    raise NotImplementedError("write your pallas kernel here")


# speedup vs baseline: 8.3933x; 8.3933x over previous
import functools

import jax
import jax.numpy as jnp
from jax import lax
from jax.experimental import pallas as pl
from jax.experimental.pallas import tpu as pltpu
from jax.experimental.pallas import tpu_sc as plsc

N = 50000
E = 800000
EBLK = 128
NBLK = E // EBLK
NCORE = 2
NSUB = 16
ROWS_MAIN = 3128
ROWS_LAST = N - (NSUB - 1) * ROWS_MAIN
SPAN = N // NSUB
SCHUNK = 625
BM = 1000
GRID = N // BM


def _mesh():
    return plsc.VectorSubcoreMesh(
        core_axis_name="c", subcore_axis_name="s",
        num_cores=NCORE, num_subcores=NSUB)


def _span_copy2(sid, src_fn, via_fn, dst_fn):
    @pl.when(sid < NSUB - 1)
    def _():
        pltpu.sync_copy(src_fn(ROWS_MAIN), via_fn(ROWS_MAIN))
        pltpu.sync_copy(via_fn(ROWS_MAIN), dst_fn(ROWS_MAIN))

    @pl.when(sid == NSUB - 1)
    def _():
        pltpu.sync_copy(src_fn(ROWS_LAST), via_fn(ROWS_LAST))
        pltpu.sync_copy(via_fn(ROWS_LAST), dst_fn(ROWS_LAST))


def _make_gcn_scatter(Dc, passes_per_core):
    n_out = max(p[1] for ps in passes_per_core for p in ps) + 1

    @functools.partial(
        pl.kernel,
        out_type=jax.ShapeDtypeStruct((n_out * N, Dc), jnp.float32),
        mesh=_mesh(),
        scratch_types=[
            pltpu.VMEM_SHARED((N, Dc), jnp.float32),
            pltpu.VMEM((EBLK,), jnp.int32),
            pltpu.VMEM((EBLK,), jnp.int32),
            pltpu.VMEM((EBLK,), jnp.int32),
            pltpu.VMEM((EBLK, Dc), jnp.float32),
            pltpu.VMEM((SCHUNK, Dc), jnp.float32),
            pltpu.SemaphoreType.DMA,
        ],
        compiler_params=pltpu.CompilerParams(use_tc_tiling_on_sc=False),
    )
    def k(row2d, col2d, zeros_h, y_flat, out, zsp, rowbuf, colbuf, rowbuf2,
          gbuf, zvbuf, sem):
        cid = lax.axis_index("c")
        sid = lax.axis_index("s")

        for core in range(NCORE):
            @pl.when(cid == core)
            def _(core=core):
                for (y_chunk, out_idx, blk_lo, blk_hi) in passes_per_core[core]:
                    pltpu.sync_copy(zeros_h, zvbuf)
                    for j in range(SPAN // SCHUNK):
                        pltpu.sync_copy(
                            zvbuf,
                            zsp.at[pl.ds(sid * SPAN + j * SCHUNK, SCHUNK)])
                    plsc.subcore_barrier()

                    off = jnp.int32(y_chunk * N)
                    n_iter = -(-(blk_hi - blk_lo) // NSUB)

                    def body(i, _, blk_lo=blk_lo, blk_hi=blk_hi, off=off):
                        blk = blk_lo + sid + NSUB * i

                        @pl.when(blk < blk_hi)
                        def _():
                            pltpu.sync_copy(row2d.at[blk], rowbuf)
                            pltpu.sync_copy(col2d.at[blk], colbuf)
                            for j in range(EBLK // 16):
                                sl = pl.ds(j * 16, 16)
                                rowbuf2[sl] = rowbuf[sl] + off
                            pltpu.async_copy(
                                y_flat.at[rowbuf2], gbuf, sem).wait()
                            pltpu.sync_copy(gbuf, zsp.at[colbuf], add=True)
                        return 0

                    lax.fori_loop(0, n_iter, body, 0)
                    plsc.subcore_barrier()

                    base = out_idx * N + sid * SPAN
                    for j in range(SPAN // SCHUNK):
                        pltpu.sync_copy(
                            zsp.at[pl.ds(sid * SPAN + j * SCHUNK, SCHUNK)],
                            zvbuf)
                        pltpu.sync_copy(
                            zvbuf, out.at[pl.ds(base + j * SCHUNK, SCHUNK)])
                    plsc.subcore_barrier()

    return k


_sc_l1 = _make_gcn_scatter(32, [
    [(0, 0, 0, NBLK), (1, 1, 0, NBLK)],
    [(2, 2, 0, NBLK), (3, 3, 0, NBLK)],
])
_sc_l2 = _make_gcn_scatter(32, [
    [(0, 0, 0, NBLK)],
    [(1, 1, 0, NBLK)],
])
_sc_l3 = _make_gcn_scatter(8, [
    [(0, 0, 0, NBLK // 2)],
    [(0, 1, NBLK // 2, NBLK)],
])


@functools.partial(
    pl.kernel,
    out_type=jax.ShapeDtypeStruct((NCORE * N,), jnp.float32),
    mesh=_mesh(),
    scratch_types=[
        pltpu.VMEM_SHARED((N,), jnp.float32),
        pltpu.VMEM((EBLK,), jnp.int32),
        pltpu.VMEM((EBLK,), jnp.float32),
        pltpu.VMEM((ROWS_MAIN,), jnp.float32),
    ],
    compiler_params=pltpu.CompilerParams(use_tc_tiling_on_sc=False),
)
def _deg_kernel(col2d, zeros_h, out, zsp, colbuf, ones_v, zvbuf):
    cid = lax.axis_index("c")
    sid = lax.axis_index("s")
    for j in range(EBLK // 16):
        ones_v[pl.ds(j * 16, 16)] = jnp.ones((16,), jnp.float32)

    for core in range(NCORE):
        @pl.when(cid == core)
        def _(core=core):
            _span_copy2(
                sid,
                lambda n: zeros_h.at[pl.ds(0, n)],
                lambda n: zvbuf.at[pl.ds(0, n)],
                lambda n: zsp.at[pl.ds(sid * ROWS_MAIN, n)])
            plsc.subcore_barrier()

            blk_lo = core * (NBLK // 2)
            blk_hi = (core + 1) * (NBLK // 2)
            n_iter = -(-(blk_hi - blk_lo) // NSUB)

            def body(i, _, blk_lo=blk_lo, blk_hi=blk_hi):
                blk = blk_lo + sid + NSUB * i

                @pl.when(blk < blk_hi)
                def _():
                    pltpu.sync_copy(col2d.at[blk], colbuf)
                    pltpu.sync_copy(ones_v, zsp.at[colbuf], add=True)
                return 0

            lax.fori_loop(0, n_iter, body, 0)
            plsc.subcore_barrier()

            base = core * N + sid * ROWS_MAIN
            _span_copy2(
                sid,
                lambda n: zsp.at[pl.ds(sid * ROWS_MAIN, n)],
                lambda n: zvbuf.at[pl.ds(0, n)],
                lambda n, base=base: out.at[pl.ds(base, n)])


def _dot(a, b):
    return lax.dot_general(a, b, (((1,), (0,)), ((), ())),
                           preferred_element_type=jnp.float32)


def _mmA_body(x_ref, w1_ref, dega_ref, degb_ref, y_ref, dis_ref):
    deg = dega_ref[:, :] + degb_ref[:, :] + 1.0
    dis = lax.rsqrt(deg)
    h = _dot(x_ref[:, :], w1_ref[:, :])
    y = h * dis
    for c in range(4):
        y_ref[c] = y[:, c * 32:(c + 1) * 32]
    dis_ref[:, :] = dis


def _mmB_body(z_ref, y_ref, dis_ref, b1_ref, w2_ref, y2_ref):
    dis = dis_ref[:, :]
    acc = jnp.zeros((BM, 64), jnp.float32)
    for c in range(4):
        o = jnp.maximum(dis * (z_ref[c] + y_ref[c]) + b1_ref[c], 0.0)
        acc = acc + _dot(o, w2_ref[c])
    y2 = acc * dis
    for d in range(2):
        y2_ref[d] = y2[:, d * 32:(d + 1) * 32]


def _mmC_body(z_ref, y_ref, dis_ref, b2_ref, w3_ref, y3_ref):
    dis = dis_ref[:, :]
    acc = jnp.zeros((BM, 8), jnp.float32)
    for c in range(2):
        o = jnp.maximum(dis * (z_ref[c] + y_ref[c]) + b2_ref[c], 0.0)
        acc = acc + _dot(o, w3_ref[c])
    y3_ref[:, :] = acc * dis


def _mmD_body(z_ref, y_ref, dis_ref, b3_ref, out_ref):
    dis = dis_ref[:, :]
    out_ref[:, :] = jnp.maximum(
        dis * (z_ref[0] + z_ref[1] + y_ref[:, :]) + b3_ref[:, :], 0.0)


def kernel(x, edge_index, W1, b1, W2, b2, W3, b3):
    row2d = edge_index[0].astype(jnp.int32).reshape(NBLK, EBLK)
    col2d = edge_index[1].astype(jnp.int32).reshape(NBLK, EBLK)
    zer32 = jnp.zeros((SCHUNK, 32), jnp.float32)
    zer8 = jnp.zeros((SCHUNK, 8), jnp.float32)
    zer1 = jnp.zeros((ROWS_MAIN,), jnp.float32)

    W1p = jnp.pad(W1, ((0, 0), (0, 28)))
    b1p = jnp.pad(b1, (0, 28))
    W2p = jnp.pad(W2, ((0, 28), (0, 14)))
    b2p = jnp.pad(b2, (0, 14))
    W3p = jnp.pad(W3, ((0, 14), (0, 1)))
    b3p = jnp.pad(b3, (0, 1))

    degp = _deg_kernel(col2d, zer1)
    dega = degp[:N].reshape(N, 1)
    degb = degp[N:].reshape(N, 1)

    y1, dis = pl.pallas_call(
        _mmA_body,
        grid=(GRID,),
        in_specs=[
            pl.BlockSpec((BM, 1433), lambda i: (i, 0)),
            pl.BlockSpec((1433, 128), lambda i: (0, 0)),
            pl.BlockSpec((BM, 1), lambda i: (i, 0)),
            pl.BlockSpec((BM, 1), lambda i: (i, 0)),
        ],
        out_specs=[
            pl.BlockSpec((4, BM, 32), lambda i: (0, i, 0)),
            pl.BlockSpec((BM, 1), lambda i: (i, 0)),
        ],
        out_shape=[
            jax.ShapeDtypeStruct((4, N, 32), jnp.float32),
            jax.ShapeDtypeStruct((N, 1), jnp.float32),
        ],
    )(x, W1p, dega, degb)

    z1 = _sc_l1(row2d, col2d, zer32, y1.reshape(4 * N, 32)).reshape(4, N, 32)

    y2 = pl.pallas_call(
        _mmB_body,
        grid=(GRID,),
        in_specs=[
            pl.BlockSpec((4, BM, 32), lambda i: (0, i, 0)),
            pl.BlockSpec((4, BM, 32), lambda i: (0, i, 0)),
            pl.BlockSpec((BM, 1), lambda i: (i, 0)),
            pl.BlockSpec((4, 1, 32), lambda i: (0, 0, 0)),
            pl.BlockSpec((4, 32, 64), lambda i: (0, 0, 0)),
        ],
        out_specs=pl.BlockSpec((2, BM, 32), lambda i: (0, i, 0)),
        out_shape=jax.ShapeDtypeStruct((2, N, 32), jnp.float32),
    )(z1, y1, dis, b1p.reshape(4, 1, 32), W2p.reshape(4, 32, 64))

    z2 = _sc_l2(row2d, col2d, zer32, y2.reshape(2 * N, 32)).reshape(2, N, 32)

    y3 = pl.pallas_call(
        _mmC_body,
        grid=(GRID,),
        in_specs=[
            pl.BlockSpec((2, BM, 32), lambda i: (0, i, 0)),
            pl.BlockSpec((2, BM, 32), lambda i: (0, i, 0)),
            pl.BlockSpec((BM, 1), lambda i: (i, 0)),
            pl.BlockSpec((2, 1, 32), lambda i: (0, 0, 0)),
            pl.BlockSpec((2, 32, 8), lambda i: (0, 0, 0)),
        ],
        out_specs=pl.BlockSpec((BM, 8), lambda i: (i, 0)),
        out_shape=jax.ShapeDtypeStruct((N, 8), jnp.float32),
    )(z2, y2, dis, b2p.reshape(2, 1, 32), W3p.reshape(2, 32, 8))

    z3 = _sc_l3(row2d, col2d, zer8, y3)

    out = pl.pallas_call(
        _mmD_body,
        grid=(GRID,),
        in_specs=[
            pl.BlockSpec((2, BM, 8), lambda i: (0, i, 0)),
            pl.BlockSpec((BM, 8), lambda i: (i, 0)),
            pl.BlockSpec((BM, 1), lambda i: (i, 0)),
            pl.BlockSpec((1, 8), lambda i: (0, 0)),
        ],
        out_specs=pl.BlockSpec((BM, 8), lambda i: (i, 0)),
        out_shape=jax.ShapeDtypeStruct((N, 8), jnp.float32),
    )(z3.reshape(2, N, 8), y3, dis, b3p.reshape(1, 8))

    return out[:, :7]


# speedup vs baseline: 11.4701x; 1.3666x over previous
import functools

import jax
import jax.numpy as jnp
from jax import lax
from jax.experimental import pallas as pl
from jax.experimental.pallas import tpu as pltpu
from jax.experimental.pallas import tpu_sc as plsc

N = 50000
E = 800000
EBLK = 128
EP = 819200
NBLK = EP // EBLK
NPAD = N + 8
NCORE = 2
NSUB = 16
ROWS_MAIN = 3128
ROWS_LAST = N - (NSUB - 1) * ROWS_MAIN
SPAN = N // NSUB
SCHUNK = 125
BM = 1000
GRID = N // BM


def _mesh():
    return plsc.VectorSubcoreMesh(
        core_axis_name="c", subcore_axis_name="s",
        num_cores=NCORE, num_subcores=NSUB)


def _span_copy2(sid, src_fn, via_fn, dst_fn):
    @pl.when(sid < NSUB - 1)
    def _():
        pltpu.sync_copy(src_fn(ROWS_MAIN), via_fn(ROWS_MAIN))
        pltpu.sync_copy(via_fn(ROWS_MAIN), dst_fn(ROWS_MAIN))

    @pl.when(sid == NSUB - 1)
    def _():
        pltpu.sync_copy(src_fn(ROWS_LAST), via_fn(ROWS_LAST))
        pltpu.sync_copy(via_fn(ROWS_LAST), dst_fn(ROWS_LAST))


def _make_gcn_scatter(Dc, passes_per_core):
    n_out = max(p[1] for ps in passes_per_core for p in ps) + 1
    GMAX = 16

    @functools.partial(
        pl.kernel,
        out_type=jax.ShapeDtypeStruct((n_out * N, Dc), jnp.float32),
        mesh=_mesh(),
        scratch_types=[
            pltpu.VMEM_SHARED((NPAD, Dc), jnp.float32),
            pltpu.VMEM((GMAX, EBLK), jnp.int32),
            pltpu.VMEM((GMAX, EBLK), jnp.int32),
            pltpu.VMEM((EBLK, Dc), jnp.float32),
            pltpu.VMEM((EBLK, Dc), jnp.float32),
            pltpu.VMEM((SCHUNK, Dc), jnp.float32),
            pltpu.SemaphoreType.DMA,
            pltpu.SemaphoreType.DMA,
        ],
        compiler_params=pltpu.CompilerParams(use_tc_tiling_on_sc=False),
    )
    def k(row2d, col2d, zeros_h, y_flat, out, zsp, rowbuf, colbuf,
          gbuf0, gbuf1, zvbuf, sem0, sem1):
        cid = lax.axis_index("c")
        sid = lax.axis_index("s")
        gbufs = (gbuf0, gbuf1)
        sems = (sem0, sem1)

        for core in range(NCORE):
            @pl.when(cid == core)
            def _(core=core):
                for (y_chunk, out_idx, blk_lo, blk_hi) in passes_per_core[core]:
                    pltpu.sync_copy(zeros_h, zvbuf)
                    for j in range(SPAN // SCHUNK):
                        pltpu.sync_copy(
                            zvbuf,
                            zsp.at[pl.ds(sid * SPAN + j * SCHUNK, SCHUNK)])
                    plsc.subcore_barrier()

                    off = jnp.int32(y_chunk * N)
                    bpt = (blk_hi - blk_lo) // NSUB
                    G = GMAX if bpt % GMAX == 0 else 8
                    n_groups = bpt // G
                    base_blk = blk_lo + sid * bpt

                    def group(g, _, base_blk=base_blk, off=off, G=G,
                              y_chunk=y_chunk):
                        g0 = base_blk + g * G
                        pltpu.sync_copy(row2d.at[pl.ds(g0, G)],
                                        rowbuf.at[pl.ds(0, G)])
                        pltpu.sync_copy(col2d.at[pl.ds(g0, G)],
                                        colbuf.at[pl.ds(0, G)])
                        if y_chunk != 0:
                            for j in range(G):
                                for jj in range(EBLK // 16):
                                    sl = pl.ds(jj * 16, 16)
                                    rowbuf[j, sl] = rowbuf[j, sl] + off
                        descs = [None, None]
                        descs[0] = pltpu.async_copy(
                            y_flat.at[rowbuf.at[0]], gbufs[0], sems[0])
                        for j in range(G):
                            if j + 1 < G:
                                b = (j + 1) % 2
                                descs[b] = pltpu.async_copy(
                                    y_flat.at[rowbuf.at[j + 1]],
                                    gbufs[b], sems[b])
                            descs[j % 2].wait()
                            pltpu.sync_copy(gbufs[j % 2],
                                            zsp.at[colbuf.at[j]], add=True)
                        return 0

                    lax.fori_loop(0, n_groups, group, 0)
                    plsc.subcore_barrier()

                    base = out_idx * N + sid * SPAN
                    for j in range(SPAN // SCHUNK):
                        pltpu.sync_copy(
                            zsp.at[pl.ds(sid * SPAN + j * SCHUNK, SCHUNK)],
                            zvbuf)
                        pltpu.sync_copy(
                            zvbuf, out.at[pl.ds(base + j * SCHUNK, SCHUNK)])
                    plsc.subcore_barrier()

    return k


_sc_l1 = _make_gcn_scatter(32, [
    [(0, 0, 0, NBLK), (1, 1, 0, NBLK)],
    [(2, 2, 0, NBLK), (3, 3, 0, NBLK)],
])
_sc_l2 = _make_gcn_scatter(32, [
    [(0, 0, 0, NBLK)],
    [(1, 1, 0, NBLK)],
])
_sc_l3 = _make_gcn_scatter(8, [
    [(0, 0, 0, NBLK // 2)],
    [(0, 1, NBLK // 2, NBLK)],
])


@functools.partial(
    pl.kernel,
    out_type=jax.ShapeDtypeStruct((NCORE * N,), jnp.float32),
    mesh=_mesh(),
    scratch_types=[
        pltpu.VMEM_SHARED((NPAD,), jnp.float32),
        pltpu.VMEM((EBLK,), jnp.int32),
        pltpu.VMEM((EBLK,), jnp.float32),
        pltpu.VMEM((ROWS_MAIN,), jnp.float32),
    ],
    compiler_params=pltpu.CompilerParams(use_tc_tiling_on_sc=False),
)
def _deg_kernel(col2d, zeros_h, out, zsp, colbuf, ones_v, zvbuf):
    cid = lax.axis_index("c")
    sid = lax.axis_index("s")
    for j in range(EBLK // 16):
        ones_v[pl.ds(j * 16, 16)] = jnp.ones((16,), jnp.float32)

    for core in range(NCORE):
        @pl.when(cid == core)
        def _(core=core):
            _span_copy2(
                sid,
                lambda n: zeros_h.at[pl.ds(0, n)],
                lambda n: zvbuf.at[pl.ds(0, n)],
                lambda n: zsp.at[pl.ds(sid * ROWS_MAIN, n)])
            plsc.subcore_barrier()

            blk_lo = core * (NBLK // 2)
            n_iter = (NBLK // 2) // NSUB

            def body(i, _, blk_lo=blk_lo):
                blk = blk_lo + sid + NSUB * i
                pltpu.sync_copy(col2d.at[blk], colbuf)
                pltpu.sync_copy(ones_v, zsp.at[colbuf], add=True)
                return 0

            lax.fori_loop(0, n_iter, body, 0)
            plsc.subcore_barrier()

            base = core * N + sid * ROWS_MAIN
            _span_copy2(
                sid,
                lambda n: zsp.at[pl.ds(sid * ROWS_MAIN, n)],
                lambda n: zvbuf.at[pl.ds(0, n)],
                lambda n, base=base: out.at[pl.ds(base, n)])


def _dot(a, b):
    return lax.dot_general(a, b, (((1,), (0,)), ((), ())),
                           preferred_element_type=jnp.float32)


def _mmA_body(x_ref, w1_ref, dega_ref, degb_ref, y_ref, dis_ref):
    deg = dega_ref[:, :] + degb_ref[:, :] + 1.0
    dis = lax.rsqrt(deg)
    h = _dot(x_ref[:, :], w1_ref[:, :])
    y = h * dis
    for c in range(4):
        y_ref[c] = y[:, c * 32:(c + 1) * 32]
    dis_ref[:, :] = dis


def _mmB_body(z_ref, y_ref, dis_ref, b1_ref, w2_ref, y2_ref):
    dis = dis_ref[:, :]
    acc = jnp.zeros((BM, 64), jnp.float32)
    for c in range(4):
        o = jnp.maximum(dis * (z_ref[c] + y_ref[c]) + b1_ref[c], 0.0)
        acc = acc + _dot(o, w2_ref[c])
    y2 = acc * dis
    for d in range(2):
        y2_ref[d] = y2[:, d * 32:(d + 1) * 32]


def _mmC_body(z_ref, y_ref, dis_ref, b2_ref, w3_ref, y3_ref):
    dis = dis_ref[:, :]
    acc = jnp.zeros((BM, 8), jnp.float32)
    for c in range(2):
        o = jnp.maximum(dis * (z_ref[c] + y_ref[c]) + b2_ref[c], 0.0)
        acc = acc + _dot(o, w3_ref[c])
    y3_ref[:, :] = acc * dis


def _mmD_body(z_ref, y_ref, dis_ref, b3_ref, out_ref):
    dis = dis_ref[:, :]
    out_ref[:, :] = jnp.maximum(
        dis * (z_ref[0] + z_ref[1] + y_ref[:, :]) + b3_ref[:, :], 0.0)


def kernel(x, edge_index, W1, b1, W2, b2, W3, b3):
    row = edge_index[0].astype(jnp.int32)
    col = edge_index[1].astype(jnp.int32)
    row2d = jnp.concatenate(
        [row, jnp.zeros((EP - E,), jnp.int32)]).reshape(NBLK, EBLK)
    col2d = jnp.concatenate(
        [col, jnp.full((EP - E,), N, jnp.int32)]).reshape(NBLK, EBLK)
    zer32 = jnp.zeros((SCHUNK, 32), jnp.float32)
    zer8 = jnp.zeros((SCHUNK, 8), jnp.float32)
    zer1 = jnp.zeros((ROWS_MAIN,), jnp.float32)

    W1p = jnp.pad(W1, ((0, 0), (0, 28)))
    b1p = jnp.pad(b1, (0, 28))
    W2p = jnp.pad(W2, ((0, 28), (0, 14)))
    b2p = jnp.pad(b2, (0, 14))
    W3p = jnp.pad(W3, ((0, 14), (0, 1)))
    b3p = jnp.pad(b3, (0, 1))

    degp = _deg_kernel(col2d, zer1)
    dega = degp[:N].reshape(N, 1)
    degb = degp[N:].reshape(N, 1)

    y1, dis = pl.pallas_call(
        _mmA_body,
        grid=(GRID,),
        in_specs=[
            pl.BlockSpec((BM, 1433), lambda i: (i, 0)),
            pl.BlockSpec((1433, 128), lambda i: (0, 0)),
            pl.BlockSpec((BM, 1), lambda i: (i, 0)),
            pl.BlockSpec((BM, 1), lambda i: (i, 0)),
        ],
        out_specs=[
            pl.BlockSpec((4, BM, 32), lambda i: (0, i, 0)),
            pl.BlockSpec((BM, 1), lambda i: (i, 0)),
        ],
        out_shape=[
            jax.ShapeDtypeStruct((4, N, 32), jnp.float32),
            jax.ShapeDtypeStruct((N, 1), jnp.float32),
        ],
    )(x, W1p, dega, degb)

    z1 = _sc_l1(row2d, col2d, zer32, y1.reshape(4 * N, 32)).reshape(4, N, 32)

    y2 = pl.pallas_call(
        _mmB_body,
        grid=(GRID,),
        in_specs=[
            pl.BlockSpec((4, BM, 32), lambda i: (0, i, 0)),
            pl.BlockSpec((4, BM, 32), lambda i: (0, i, 0)),
            pl.BlockSpec((BM, 1), lambda i: (i, 0)),
            pl.BlockSpec((4, 1, 32), lambda i: (0, 0, 0)),
            pl.BlockSpec((4, 32, 64), lambda i: (0, 0, 0)),
        ],
        out_specs=pl.BlockSpec((2, BM, 32), lambda i: (0, i, 0)),
        out_shape=jax.ShapeDtypeStruct((2, N, 32), jnp.float32),
    )(z1, y1, dis, b1p.reshape(4, 1, 32), W2p.reshape(4, 32, 64))

    z2 = _sc_l2(row2d, col2d, zer32, y2.reshape(2 * N, 32)).reshape(2, N, 32)

    y3 = pl.pallas_call(
        _mmC_body,
        grid=(GRID,),
        in_specs=[
            pl.BlockSpec((2, BM, 32), lambda i: (0, i, 0)),
            pl.BlockSpec((2, BM, 32), lambda i: (0, i, 0)),
            pl.BlockSpec((BM, 1), lambda i: (i, 0)),
            pl.BlockSpec((2, 1, 32), lambda i: (0, 0, 0)),
            pl.BlockSpec((2, 32, 8), lambda i: (0, 0, 0)),
        ],
        out_specs=pl.BlockSpec((BM, 8), lambda i: (i, 0)),
        out_shape=jax.ShapeDtypeStruct((N, 8), jnp.float32),
    )(z2, y2, dis, b2p.reshape(2, 1, 32), W3p.reshape(2, 32, 8))

    z3 = _sc_l3(row2d, col2d, zer8, y3)

    out = pl.pallas_call(
        _mmD_body,
        grid=(GRID,),
        in_specs=[
            pl.BlockSpec((2, BM, 8), lambda i: (0, i, 0)),
            pl.BlockSpec((BM, 8), lambda i: (i, 0)),
            pl.BlockSpec((BM, 1), lambda i: (i, 0)),
            pl.BlockSpec((1, 8), lambda i: (0, 0)),
        ],
        out_specs=pl.BlockSpec((BM, 8), lambda i: (i, 0)),
        out_shape=jax.ShapeDtypeStruct((N, 8), jnp.float32),
    )(z3.reshape(2, N, 8), y3, dis, b3p.reshape(1, 8))

    return out[:, :7]


# speedup vs baseline: 12.4211x; 1.0829x over previous
import functools

import jax
import jax.numpy as jnp
from jax import lax
from jax.experimental import pallas as pl
from jax.experimental.pallas import tpu as pltpu
from jax.experimental.pallas import tpu_sc as plsc

N = 50000
E = 800000
EBLK = 128
EP = 819200
NBLK = EP // EBLK
NPAD = N + 8
NCORE = 2
NSUB = 16
ROWS_MAIN = 3128
ROWS_LAST = N - (NSUB - 1) * ROWS_MAIN
SPAN = N // NSUB
SCHUNK = 125
BM = 1000
GRID = N // BM


def _mesh():
    return plsc.VectorSubcoreMesh(
        core_axis_name="c", subcore_axis_name="s",
        num_cores=NCORE, num_subcores=NSUB)


def _span_copy2(sid, src_fn, via_fn, dst_fn):
    @pl.when(sid < NSUB - 1)
    def _():
        pltpu.sync_copy(src_fn(ROWS_MAIN), via_fn(ROWS_MAIN))
        pltpu.sync_copy(via_fn(ROWS_MAIN), dst_fn(ROWS_MAIN))

    @pl.when(sid == NSUB - 1)
    def _():
        pltpu.sync_copy(src_fn(ROWS_LAST), via_fn(ROWS_LAST))
        pltpu.sync_copy(via_fn(ROWS_LAST), dst_fn(ROWS_LAST))


def _make_gcn_scatter(Dc, passes_per_core):
    n_out = max(p[1] for ps in passes_per_core for p in ps) + 1
    n_chunks = max(p[0] for ps in passes_per_core for p in ps) + 1
    GMAX = 16
    D = 3

    @functools.partial(
        pl.kernel,
        out_type=jax.ShapeDtypeStruct((n_out * N, Dc), jnp.float32),
        mesh=_mesh(),
        scratch_types=(
            [pltpu.VMEM_SHARED((NPAD, Dc), jnp.float32),
             pltpu.VMEM((GMAX, EBLK), jnp.int32),
             pltpu.VMEM((GMAX, EBLK), jnp.int32)]
            + [pltpu.VMEM((EBLK, Dc), jnp.float32) for _ in range(D)]
            + [pltpu.VMEM((SCHUNK, Dc), jnp.float32)]
            + [pltpu.SemaphoreType.DMA for _ in range(2 * D)]
        ),
        compiler_params=pltpu.CompilerParams(use_tc_tiling_on_sc=False),
    )
    def k(*refs):
        row2d, col2d, zeros_h = refs[0], refs[1], refs[2]
        ys = refs[3:3 + n_chunks]
        out = refs[3 + n_chunks]
        s = 4 + n_chunks
        zsp, rowbuf, colbuf = refs[s], refs[s + 1], refs[s + 2]
        gbufs = refs[s + 3:s + 3 + D]
        zvbuf = refs[s + 3 + D]
        gsems = refs[s + 4 + D:s + 4 + 2 * D]
        ssems = refs[s + 4 + 2 * D:s + 4 + 3 * D]
        cid = lax.axis_index("c")
        sid = lax.axis_index("s")

        for core in range(NCORE):
            @pl.when(cid == core)
            def _(core=core):
                for (y_chunk, out_idx, blk_lo, blk_hi) in passes_per_core[core]:
                    pltpu.sync_copy(zeros_h, zvbuf)
                    for j in range(SPAN // SCHUNK):
                        pltpu.sync_copy(
                            zvbuf,
                            zsp.at[pl.ds(sid * SPAN + j * SCHUNK, SCHUNK)])
                    plsc.subcore_barrier()

                    y_ref = ys[y_chunk]
                    bpt = (blk_hi - blk_lo) // NSUB
                    G = GMAX if bpt % GMAX == 0 else 8
                    n_groups = bpt // G
                    base_blk = blk_lo + sid * bpt

                    def group(g, _, base_blk=base_blk, G=G, y_ref=y_ref):
                        g0 = base_blk + g * G
                        pltpu.sync_copy(row2d.at[pl.ds(g0, G)],
                                        rowbuf.at[pl.ds(0, G)])
                        pltpu.sync_copy(col2d.at[pl.ds(g0, G)],
                                        colbuf.at[pl.ds(0, G)])
                        gd = [None] * D
                        sd = [None] * D
                        for j in range(G):
                            b = j % D
                            if j >= D:
                                sd[b].wait()
                            gd[b] = pltpu.async_copy(
                                y_ref.at[rowbuf.at[j]], gbufs[b], gsems[b])
                            if j >= D - 1:
                                jj = j - (D - 1)
                                bb = jj % D
                                gd[bb].wait()
                                sd[bb] = pltpu.async_copy(
                                    gbufs[bb], zsp.at[colbuf.at[jj]],
                                    ssems[bb], add=True)
                        for jj in range(G - D + 1, G):
                            bb = jj % D
                            gd[bb].wait()
                            sd[bb] = pltpu.async_copy(
                                gbufs[bb], zsp.at[colbuf.at[jj]],
                                ssems[bb], add=True)
                        for bb in range(D):
                            if sd[bb] is not None:
                                sd[bb].wait()
                        return 0

                    lax.fori_loop(0, n_groups, group, 0)
                    plsc.subcore_barrier()

                    base = out_idx * N + sid * SPAN
                    for j in range(SPAN // SCHUNK):
                        pltpu.sync_copy(
                            zsp.at[pl.ds(sid * SPAN + j * SCHUNK, SCHUNK)],
                            zvbuf)
                        pltpu.sync_copy(
                            zvbuf, out.at[pl.ds(base + j * SCHUNK, SCHUNK)])
                    plsc.subcore_barrier()

    return k


_sc_l1 = _make_gcn_scatter(32, [
    [(0, 0, 0, NBLK), (1, 1, 0, NBLK)],
    [(2, 2, 0, NBLK), (3, 3, 0, NBLK)],
])
_sc_l2 = _make_gcn_scatter(32, [
    [(0, 0, 0, NBLK)],
    [(1, 1, 0, NBLK)],
])
_sc_l3 = _make_gcn_scatter(8, [
    [(0, 0, 0, NBLK // 2)],
    [(0, 1, NBLK // 2, NBLK)],
])


@functools.partial(
    pl.kernel,
    out_type=jax.ShapeDtypeStruct((NCORE * N,), jnp.float32),
    mesh=_mesh(),
    scratch_types=[
        pltpu.VMEM_SHARED((NPAD,), jnp.float32),
        pltpu.VMEM((EBLK,), jnp.int32),
        pltpu.VMEM((EBLK,), jnp.float32),
        pltpu.VMEM((ROWS_MAIN,), jnp.float32),
    ],
    compiler_params=pltpu.CompilerParams(use_tc_tiling_on_sc=False),
)
def _deg_kernel(col2d, zeros_h, out, zsp, colbuf, ones_v, zvbuf):
    cid = lax.axis_index("c")
    sid = lax.axis_index("s")
    for j in range(EBLK // 16):
        ones_v[pl.ds(j * 16, 16)] = jnp.ones((16,), jnp.float32)

    for core in range(NCORE):
        @pl.when(cid == core)
        def _(core=core):
            _span_copy2(
                sid,
                lambda n: zeros_h.at[pl.ds(0, n)],
                lambda n: zvbuf.at[pl.ds(0, n)],
                lambda n: zsp.at[pl.ds(sid * ROWS_MAIN, n)])
            plsc.subcore_barrier()

            blk_lo = core * (NBLK // 2)
            n_iter = (NBLK // 2) // NSUB

            def body(i, _, blk_lo=blk_lo):
                blk = blk_lo + sid + NSUB * i
                pltpu.sync_copy(col2d.at[blk], colbuf)
                pltpu.sync_copy(ones_v, zsp.at[colbuf], add=True)
                return 0

            lax.fori_loop(0, n_iter, body, 0)
            plsc.subcore_barrier()

            base = core * N + sid * ROWS_MAIN
            _span_copy2(
                sid,
                lambda n: zsp.at[pl.ds(sid * ROWS_MAIN, n)],
                lambda n: zvbuf.at[pl.ds(0, n)],
                lambda n, base=base: out.at[pl.ds(base, n)])


def _dot(a, b):
    return lax.dot_general(a, b, (((1,), (0,)), ((), ())),
                           preferred_element_type=jnp.float32)


def _mmA_body(x_ref, w1_ref, dega_ref, degb_ref, y0, y1, y2, y3, dis_ref):
    deg = dega_ref[:, :] + degb_ref[:, :] + 1.0
    dis = lax.rsqrt(deg)
    h = _dot(x_ref[:, :], w1_ref[:, :])
    y = h * dis
    for c, y_ref in enumerate((y0, y1, y2, y3)):
        y_ref[:, :] = y[:, c * 32:(c + 1) * 32]
    dis_ref[:, :] = dis


def _mmB_body(z_ref, y0, y1, y2, y3, dis_ref, b1_ref, w2_ref, o0, o1):
    dis = dis_ref[:, :]
    acc = jnp.zeros((BM, 64), jnp.float32)
    for c, y_ref in enumerate((y0, y1, y2, y3)):
        o = jnp.maximum(dis * (z_ref[c] + y_ref[:, :]) + b1_ref[c], 0.0)
        acc = acc + _dot(o, w2_ref[c])
    y2o = acc * dis
    for d, o_ref in enumerate((o0, o1)):
        o_ref[:, :] = y2o[:, d * 32:(d + 1) * 32]


def _mmC_body(z_ref, y0, y1, dis_ref, b2_ref, w3_ref, y3_ref):
    dis = dis_ref[:, :]
    acc = jnp.zeros((BM, 8), jnp.float32)
    for c, y_ref in enumerate((y0, y1)):
        o = jnp.maximum(dis * (z_ref[c] + y_ref[:, :]) + b2_ref[c], 0.0)
        acc = acc + _dot(o, w3_ref[c])
    y3_ref[:, :] = acc * dis


def _mmD_body(z_ref, y_ref, dis_ref, b3_ref, out_ref):
    dis = dis_ref[:, :]
    out_ref[:, :] = jnp.maximum(
        dis * (z_ref[0] + z_ref[1] + y_ref[:, :]) + b3_ref[:, :], 0.0)


def kernel(x, edge_index, W1, b1, W2, b2, W3, b3):
    row = edge_index[0].astype(jnp.int32)
    col = edge_index[1].astype(jnp.int32)
    row2d = jnp.concatenate(
        [row, jnp.zeros((EP - E,), jnp.int32)]).reshape(NBLK, EBLK)
    col2d = jnp.concatenate(
        [col, jnp.full((EP - E,), N, jnp.int32)]).reshape(NBLK, EBLK)
    zer32 = jnp.zeros((SCHUNK, 32), jnp.float32)
    zer8 = jnp.zeros((SCHUNK, 8), jnp.float32)
    zer1 = jnp.zeros((ROWS_MAIN,), jnp.float32)

    W1p = jnp.pad(W1, ((0, 0), (0, 28)))
    b1p = jnp.pad(b1, (0, 28))
    W2p = jnp.pad(W2, ((0, 28), (0, 14)))
    b2p = jnp.pad(b2, (0, 14))
    W3p = jnp.pad(W3, ((0, 14), (0, 1)))
    b3p = jnp.pad(b3, (0, 1))

    degp = _deg_kernel(col2d, zer1)
    dega = degp[:N].reshape(N, 1)
    degb = degp[N:].reshape(N, 1)

    bspec = pl.BlockSpec((BM, 32), lambda i: (i, 0))
    yshape = jax.ShapeDtypeStruct((N, 32), jnp.float32)

    *y1s, dis = pl.pallas_call(
        _mmA_body,
        grid=(GRID,),
        in_specs=[
            pl.BlockSpec((BM, 1433), lambda i: (i, 0)),
            pl.BlockSpec((1433, 128), lambda i: (0, 0)),
            pl.BlockSpec((BM, 1), lambda i: (i, 0)),
            pl.BlockSpec((BM, 1), lambda i: (i, 0)),
        ],
        out_specs=[bspec, bspec, bspec, bspec,
                   pl.BlockSpec((BM, 1), lambda i: (i, 0))],
        out_shape=[yshape, yshape, yshape, yshape,
                   jax.ShapeDtypeStruct((N, 1), jnp.float32)],
    )(x, W1p, dega, degb)

    z1 = _sc_l1(row2d, col2d, zer32, *y1s).reshape(4, N, 32)

    y2s = pl.pallas_call(
        _mmB_body,
        grid=(GRID,),
        in_specs=[
            pl.BlockSpec((4, BM, 32), lambda i: (0, i, 0)),
            bspec, bspec, bspec, bspec,
            pl.BlockSpec((BM, 1), lambda i: (i, 0)),
            pl.BlockSpec((4, 1, 32), lambda i: (0, 0, 0)),
            pl.BlockSpec((4, 32, 64), lambda i: (0, 0, 0)),
        ],
        out_specs=[bspec, bspec],
        out_shape=[yshape, yshape],
    )(z1, *y1s, dis, b1p.reshape(4, 1, 32), W2p.reshape(4, 32, 64))

    z2 = _sc_l2(row2d, col2d, zer32, *y2s).reshape(2, N, 32)

    y3 = pl.pallas_call(
        _mmC_body,
        grid=(GRID,),
        in_specs=[
            pl.BlockSpec((2, BM, 32), lambda i: (0, i, 0)),
            bspec, bspec,
            pl.BlockSpec((BM, 1), lambda i: (i, 0)),
            pl.BlockSpec((2, 1, 32), lambda i: (0, 0, 0)),
            pl.BlockSpec((2, 32, 8), lambda i: (0, 0, 0)),
        ],
        out_specs=pl.BlockSpec((BM, 8), lambda i: (i, 0)),
        out_shape=jax.ShapeDtypeStruct((N, 8), jnp.float32),
    )(z2, *y2s, dis, b2p.reshape(2, 1, 32), W3p.reshape(2, 32, 8))

    z3 = _sc_l3(row2d, col2d, zer8, y3)

    out = pl.pallas_call(
        _mmD_body,
        grid=(GRID,),
        in_specs=[
            pl.BlockSpec((2, BM, 8), lambda i: (0, i, 0)),
            pl.BlockSpec((BM, 8), lambda i: (i, 0)),
            pl.BlockSpec((BM, 1), lambda i: (i, 0)),
            pl.BlockSpec((1, 8), lambda i: (0, 0)),
        ],
        out_specs=pl.BlockSpec((BM, 8), lambda i: (i, 0)),
        out_shape=jax.ShapeDtypeStruct((N, 8), jnp.float32),
    )(z3.reshape(2, N, 8), y3, dis, b3p.reshape(1, 8))

    return out[:, :7]
